# baseline (speedup 1.0000x reference)
import functools

import jax
import jax.numpy as jnp
from jax import lax
from jax.experimental import pallas as pl
from jax.experimental.pallas import tpu as pltpu
from jax.experimental.pallas import tpu_sc as plsc

_S, _T = 4096, 50
_D = 128
_NC, _NS = 2, 16
_NW = _NC * _NS
_SPW = _S // _NW
_CT = 2
_NCH = _T // _CT
_NBUF = 3

_mesh = plsc.VectorSubcoreMesh(core_axis_name="c", subcore_axis_name="s")


@functools.partial(
    pl.kernel,
    mesh=_mesh,
    out_type=jax.ShapeDtypeStruct((_T, _S, _D), jnp.float32),
    scratch_types=[
        pltpu.VMEM((_T, _SPW), jnp.int32),
        pltpu.VMEM((_NBUF, _CT, _SPW, _D), jnp.float32),
        [pltpu.SemaphoreType.DMA] * _NBUF,
        [pltpu.SemaphoreType.DMA] * _NBUF,
    ],
)
def _emb_lookup(table_hbm, idx_hbm, out_hbm, idx_v, rows_v, gsem, ssem):
    wid = lax.axis_index("s") * _NC + lax.axis_index("c")
    s0 = wid * _SPW
    pltpu.sync_copy(idx_hbm.at[:, pl.ds(s0, _SPW)], idx_v)

    def start_g(j, b):
        for t in range(_CT):
            pltpu.async_copy(
                table_hbm.at[idx_v.at[j * _CT + t]], rows_v.at[b, t], gsem[b])

    def wait_g(b):
        for t in range(_CT):
            pltpu.make_async_copy(
                table_hbm.at[idx_v.at[0]], rows_v.at[b, t], gsem[b]).wait()

    def start_s(j, b):
        pltpu.async_copy(
            rows_v.at[b], out_hbm.at[pl.ds(j * _CT, _CT), pl.ds(s0, _SPW)],
            ssem[b])

    def wait_s(b):
        pltpu.make_async_copy(
            rows_v.at[b], out_hbm.at[pl.ds(0, _CT), pl.ds(s0, _SPW)],
            ssem[b]).wait()

    for b in range(_NBUF - 1):
        start_g(b, b)
    wait_g(0)
    start_s(0, 0)
    start_g(_NBUF - 1, _NBUF - 1)

    _last_g = _NCH - _NBUF
    _aligned = ((_last_g) // _NBUF) * _NBUF

    @pl.loop(1, _aligned + 1, step=_NBUF)
    def _steady(i):
        for u in range(_NBUF):
            j = i + u
            b = (u + 1) % _NBUF
            bp = u % _NBUF
            wait_g(b)
            start_s(j, b)
            wait_s(bp)
            start_g(j + _NBUF - 1, bp)

    for j in range(_aligned + 1, _NCH):
        b = j % _NBUF
        bp = (j - 1) % _NBUF
        wait_g(b)
        start_s(j, b)
        if j + _NBUF - 1 < _NCH:
            wait_s(bp)
            start_g(j + _NBUF - 1, bp)
    for j in range(_NCH - _NBUF, _NCH):
        wait_s(j % _NBUF)


def kernel(token_ids, weight):
    idx_t = jnp.transpose(token_ids).astype(jnp.int32)
    out_t = _emb_lookup(weight, idx_t)
    return jnp.transpose(out_t, (1, 0, 2))


# speedup vs baseline: 1.0179x; 1.0179x over previous
import functools

import jax
import jax.numpy as jnp
from jax import lax
from jax.experimental import pallas as pl
from jax.experimental.pallas import tpu as pltpu
from jax.experimental.pallas import tpu_sc as plsc

_S, _T = 4096, 50
_D = 128
_NC, _NS = 2, 16
_NW = _NC * _NS
_SPW = _S // _NW
_CT = 1
_NCH = _T // _CT
_NBUF = 7

_mesh = plsc.VectorSubcoreMesh(core_axis_name="c", subcore_axis_name="s")


@functools.partial(
    pl.kernel,
    mesh=_mesh,
    out_type=jax.ShapeDtypeStruct((_T, _S, _D), jnp.float32),
    scratch_types=[
        pltpu.VMEM((_T, _SPW), jnp.int32),
        pltpu.VMEM((_NBUF, _CT, _SPW, _D), jnp.float32),
        [pltpu.SemaphoreType.DMA] * _NBUF,
        [pltpu.SemaphoreType.DMA] * _NBUF,
    ],
)
def _emb_lookup(table_hbm, idx_hbm, out_hbm, idx_v, rows_v, gsem, ssem):
    wid = lax.axis_index("s") * _NC + lax.axis_index("c")
    s0 = wid * _SPW
    pltpu.sync_copy(idx_hbm.at[:, pl.ds(s0, _SPW)], idx_v)

    def start_g(j, b):
        for t in range(_CT):
            pltpu.async_copy(
                table_hbm.at[idx_v.at[j * _CT + t]], rows_v.at[b, t], gsem[b])

    def wait_g(b):
        for t in range(_CT):
            pltpu.make_async_copy(
                table_hbm.at[idx_v.at[0]], rows_v.at[b, t], gsem[b]).wait()

    def start_s(j, b):
        pltpu.async_copy(
            rows_v.at[b], out_hbm.at[pl.ds(j * _CT, _CT), pl.ds(s0, _SPW)],
            ssem[b])

    def wait_s(b):
        pltpu.make_async_copy(
            rows_v.at[b], out_hbm.at[pl.ds(0, _CT), pl.ds(s0, _SPW)],
            ssem[b]).wait()

    for b in range(_NBUF - 1):
        start_g(b, b)
    wait_g(0)
    start_s(0, 0)
    start_g(_NBUF - 1, _NBUF - 1)

    _last_g = _NCH - _NBUF
    _aligned = ((_last_g) // _NBUF) * _NBUF

    @pl.loop(1, _aligned + 1, step=_NBUF)
    def _steady(i):
        for u in range(_NBUF):
            j = i + u
            b = (u + 1) % _NBUF
            bp = u % _NBUF
            wait_g(b)
            start_s(j, b)
            wait_s(bp)
            start_g(j + _NBUF - 1, bp)

    for j in range(_aligned + 1, _NCH):
        b = j % _NBUF
        bp = (j - 1) % _NBUF
        wait_g(b)
        start_s(j, b)
        if j + _NBUF - 1 < _NCH:
            wait_s(bp)
            start_g(j + _NBUF - 1, bp)
    for j in range(_NCH - _NBUF, _NCH):
        wait_s(j % _NBUF)


def kernel(token_ids, weight):
    idx_t = jnp.transpose(token_ids).astype(jnp.int32)
    out_t = _emb_lookup(weight, idx_t)
    return jnp.transpose(out_t, (1, 0, 2))
